# baseline (speedup 1.0000x reference)
import jax, jax.numpy as jnp
import numpy as np
from jax.experimental import pallas as pl


def _sqdist(src, dst):
    return jnp.sum(src ** 2, -1)[:, :, None] + jnp.sum(dst ** 2, -1)[:, None, :] - 2.0 * jnp.einsum('bnc,bmc->bnm', src, dst)


def _index_points(points, idx):
    return jax.vmap(lambda p, i: p[i])(points, idx)


def _fps(xyz, npoint):
    B, N, _ = xyz.shape

    def body(carry, _):
        distance, farthest = carry
        centroid = _index_points(xyz, farthest[:, None])
        dist = jnp.sum((xyz - centroid) ** 2, -1)
        distance = jnp.minimum(distance, dist)
        new_far = jnp.argmax(distance, -1).astype(jnp.int32)
        return (distance, new_far), farthest

    init = (jnp.full((B, N), 1e10, dtype=xyz.dtype), jnp.zeros((B,), dtype=jnp.int32))
    _, idxs = jax.lax.scan(body, init, None, length=npoint)
    return jnp.transpose(idxs)


def _qbp(radius, nsample, xyz, new_xyz):
    B, N, _ = xyz.shape
    S = new_xyz.shape[1]
    sqrdists = _sqdist(new_xyz, xyz)
    group_idx = jnp.broadcast_to(jnp.arange(N, dtype=jnp.int32), (B, S, N))
    group_idx = jnp.where(sqrdists > radius ** 2, N, group_idx)
    group_idx = jnp.sort(group_idx, axis=-1)[:, :, :nsample]
    group_first = jnp.broadcast_to(group_idx[:, :, :1], group_idx.shape)
    group_idx = jnp.where(group_idx == N, group_first, group_idx)
    return jnp.minimum(group_idx, N - 1)


def _mlp(x, layers):
    for (W, b, gamma, beta, mean, var) in layers:
        x = jnp.einsum('...i,oi->...o', x, W) + b
        x = (x - mean) / jnp.sqrt(var + 1e-5) * gamma + beta
        x = jax.nn.relu(x)
    return x


def _sa(xyz, points, layers, npoint, radius, nsample):
    pts = jnp.transpose(points, (0, 2, 1))
    fps_idx = _fps(xyz, npoint)
    new_xyz = _index_points(xyz, fps_idx)
    idx = _qbp(radius, nsample, xyz, new_xyz)
    grouped_xyz = _index_points(xyz, idx) - new_xyz[:, :, None, :]
    grouped = jnp.concatenate([grouped_xyz, _index_points(pts, idx)], axis=-1)
    out = _mlp(grouped, layers)
    new_points = jnp.max(out, axis=2)
    return new_xyz, jnp.transpose(new_points, (0, 2, 1))


def _fp(xyz1, xyz2, points1, points2, layers):
    pts2 = jnp.transpose(points2, (0, 2, 1))
    dists = _sqdist(xyz1, xyz2)
    idx = jnp.argsort(dists, axis=-1)[:, :, :3]
    d = jnp.take_along_axis(dists, idx, axis=-1)
    dist_recip = 1.0 / (d + 1e-8)
    norm = jnp.sum(dist_recip, axis=-1, keepdims=True)
    weight = dist_recip / norm
    interpolated = jnp.sum(_index_points(pts2, idx) * weight[..., None], axis=2)
    pts1 = jnp.transpose(points1, (0, 2, 1))
    new_points = jnp.concatenate([pts1, interpolated], axis=-1)
    out = _mlp(new_points, layers)
    return jnp.transpose(out, (0, 2, 1))


def _copy_kernel(x_ref, o_ref):
    o_ref[...] = x_ref[...]


def kernel(xyz, points, params):
    points_t = jnp.transpose(points, (0, 2, 1))
    l1_xyz, l1_points = _sa(xyz, points_t, params['sa1'], 1024, 0.1, 32)
    l2_xyz, l2_points = _sa(l1_xyz, l1_points, params['sa2'], 256, 0.2, 32)
    l3_xyz, l3_points = _sa(l2_xyz, l2_points, params['sa3'], 64, 0.4, 32)
    l2_points = _fp(l2_xyz, l3_xyz, l2_points, l3_points, params['fp3'])
    l1_points = _fp(l1_xyz, l2_xyz, l1_points, l2_points, params['fp2'])
    l0_points = _fp(xyz, l1_xyz, points_t, l1_points, params['fp1'])
    h = params['head']
    x = jnp.transpose(l0_points, (0, 2, 1))
    x = jnp.einsum('bni,oi->bno', x, h['conv1_W']) + h['conv1_b']
    x = (x - h['bn1_mean']) / jnp.sqrt(h['bn1_var'] + 1e-5) * h['bn1_gamma'] + h['bn1_beta']
    x = jax.nn.relu(x)
    x = jnp.einsum('bni,oi->bno', x, h['conv2_W']) + h['conv2_b']
    x = pl.pallas_call(_copy_kernel, out_shape=jax.ShapeDtypeStruct(x.shape, x.dtype))(x)
    return jnp.transpose(x, (0, 2, 1))


# speedup vs baseline: 4.1571x; 4.1571x over previous
import functools

import jax
import jax.numpy as jnp
from jax import lax
from jax.experimental import pallas as pl
from jax.experimental.pallas import tpu as pltpu
from jax.experimental.pallas import tpu_sc as plsc


def _fps_body(S, x_ref, y_ref, z_ref, idx_ref, cx_ref, cy_ref, cz_ref):
    B, R, L = x_ref.shape
    N = R * L
    lin = (lax.broadcasted_iota(jnp.int32, (B, R, L), 1) * L
           + lax.broadcasted_iota(jnp.int32, (B, R, L), 2))
    X = x_ref[...]
    Y = y_ref[...]
    Z = z_ref[...]

    def red2(v, red):
        return red(red(v, axis=2), axis=1)

    def body(t, carry):
        dist, far = carry
        mask = lin == far[:, None, None]
        cx = red2(jnp.where(mask, X, 0.0), jnp.sum)
        cy = red2(jnp.where(mask, Y, 0.0), jnp.sum)
        cz = red2(jnp.where(mask, Z, 0.0), jnp.sum)
        idx_ref[pl.ds(t, 1), :] = far[None, :]
        cx_ref[pl.ds(t, 1), :] = cx[None, :]
        cy_ref[pl.ds(t, 1), :] = cy[None, :]
        cz_ref[pl.ds(t, 1), :] = cz[None, :]
        d = ((X - cx[:, None, None]) ** 2 + (Y - cy[:, None, None]) ** 2
             + (Z - cz[:, None, None]) ** 2)
        dist = jnp.minimum(dist, d)
        m = red2(dist, jnp.max)
        li = red2(jnp.where(dist == m[:, None, None], lin, N), jnp.min)
        return dist, li

    dist0 = jnp.full((B, R, L), 1e10, dtype=jnp.float32)
    far0 = jnp.zeros((B,), dtype=jnp.int32)
    lax.fori_loop(0, S, body, (dist0, far0))


def _fps(xyz, S):
    B, N, _ = xyz.shape
    R = N // 128
    X = xyz[..., 0].reshape(B, R, 128)
    Y = xyz[..., 1].reshape(B, R, 128)
    Z = xyz[..., 2].reshape(B, R, 128)
    outs = pl.pallas_call(
        functools.partial(_fps_body, S),
        out_shape=[
            jax.ShapeDtypeStruct((S, B), jnp.int32),
            jax.ShapeDtypeStruct((S, B), jnp.float32),
            jax.ShapeDtypeStruct((S, B), jnp.float32),
            jax.ShapeDtypeStruct((S, B), jnp.float32),
        ],
    )(X, Y, Z)
    return outs[0].T


def _ballquery_body(r2, K, N, nx_ref, x_ref, out_ref):
    b = pl.program_id(0)
    A = nx_ref[0]
    X = x_ref[0]
    SB = A.shape[0]
    s2 = jnp.sum(A * A, axis=1)
    n2 = jnp.sum(X * X, axis=1)
    dots = lax.dot_general(A, X, (((1,), (1,)), ((), ())))
    d = s2[:, None] + n2[None, :] - 2.0 * dots
    iota = lax.broadcasted_iota(jnp.int32, (SB, N), 1)
    cand0 = jnp.where(d > r2, N, iota)
    lane = lax.broadcasted_iota(jnp.int32, (SB, K), 1)
    BIG = jnp.int32(1 << 30)

    def body(j, carry):
        cand, cols = carry
        mj = jnp.min(cand, axis=1)
        cols = jnp.where(lane == j, mj[:, None], cols)
        cand = jnp.where(cand == mj[:, None], BIG, cand)
        return cand, cols

    _, cols = lax.fori_loop(0, K, body, (cand0, jnp.zeros((SB, K), jnp.int32)))
    col0 = cols[:, 0:1]
    cols = jnp.where(cols >= N, col0, cols)
    cols = jnp.minimum(cols, N - 1)
    out_ref[0] = cols + b * N


def _ballquery(radius, K, xyz, new_xyz, SB):
    B, N, _ = xyz.shape
    S = new_xyz.shape[1]
    r2 = float(radius) ** 2
    return pl.pallas_call(
        functools.partial(_ballquery_body, r2, K, N),
        grid=(B, S // SB),
        in_specs=[
            pl.BlockSpec((1, SB, 3), lambda b, s: (b, s, 0)),
            pl.BlockSpec((1, N, 3), lambda b, s: (b, 0, 0)),
        ],
        out_specs=pl.BlockSpec((1, SB, K), lambda b, s: (b, s, 0)),
        out_shape=jax.ShapeDtypeStruct((B, S, K), jnp.int32),
    )(new_xyz, xyz)


def _sc_gather(table, idx):
    V, Dp = table.shape
    (Bi,) = idx.shape
    NW = 32
    bpw = Bi // NW
    CH = min(128, bpw)
    nch = bpw // CH
    mesh = plsc.VectorSubcoreMesh(core_axis_name="c", subcore_axis_name="s")

    @functools.partial(
        pl.kernel,
        out_type=jax.ShapeDtypeStruct((Bi, Dp), jnp.float32),
        scratch_types=[
            pltpu.VMEM((CH,), jnp.int32),
            pltpu.VMEM((CH, Dp), jnp.float32),
            pltpu.SemaphoreType.DMA,
        ],
        mesh=mesh,
        compiler_params=pltpu.CompilerParams(use_tc_tiling_on_sc=False),
    )
    def k(table_hbm, idx_hbm, out_hbm, idx_v, rows_v, sem):
        wid = lax.axis_index("s") * 2 + lax.axis_index("c")
        base = wid * bpw

        def body(c, carry):
            off = base + c * CH
            pltpu.sync_copy(idx_hbm.at[pl.ds(off, CH)], idx_v)
            pltpu.async_copy(table_hbm.at[idx_v], rows_v, sem).wait()
            pltpu.sync_copy(rows_v, out_hbm.at[pl.ds(off, CH)])
            return carry

        lax.fori_loop(0, nch, body, 0)

    return k(table, idx)


def _sa_mlp_body(nl, G, K, Dp, gath_ref, cen_ref, *args):
    w = [args[3 * i] for i in range(nl)]
    a = [args[3 * i + 1] for i in range(nl)]
    d = [args[3 * i + 2] for i in range(nl)]
    out_ref = args[3 * nl]
    g = gath_ref[...]
    c = cen_ref[...]
    cpad = jnp.concatenate(
        [c[:, :3].reshape(G, 1, 3), jnp.zeros((G, 1, Dp - 3), jnp.float32)],
        axis=2)
    h = (g - cpad).reshape(G * K, Dp)
    for i in range(nl):
        h = jnp.dot(h, w[i][...], preferred_element_type=jnp.float32)
        h = jnp.maximum(h * a[i][...] + d[i][...], 0.0)
    C = h.shape[-1]
    out_ref[...] = jnp.max(h.reshape(G, K, C), axis=1)


def _sa_mlp(gathered, centers, layers, G):
    BS, K, Dp = gathered.shape
    nl = len(layers)
    Cout = layers[-1][0].shape[1]
    in_specs = [
        pl.BlockSpec((G, K, Dp), lambda i: (i, 0, 0)),
        pl.BlockSpec((G, 8), lambda i: (i, 0)),
    ]
    ops = []
    for (wt, av, dv) in layers:
        in_specs.append(pl.BlockSpec(wt.shape, lambda i: (0, 0)))
        in_specs.append(pl.BlockSpec(av.shape, lambda i: (0, 0)))
        in_specs.append(pl.BlockSpec(dv.shape, lambda i: (0, 0)))
        ops.extend([wt, av, dv])
    return pl.pallas_call(
        functools.partial(_sa_mlp_body, nl, G, K, Dp),
        grid=(BS // G,),
        in_specs=in_specs,
        out_specs=pl.BlockSpec((G, Cout), lambda i: (i, 0)),
        out_shape=jax.ShapeDtypeStruct((BS, Cout), jnp.float32),
    )(gathered, centers, *ops)


def _fp_select_body(S2, x1_ref, x2_ref, idx_ref, d_ref):
    A1 = x1_ref[0]
    A2 = x2_ref[0]
    G = A1.shape[0]
    s1 = jnp.sum(A1 * A1, axis=1)
    s2 = jnp.sum(A2 * A2, axis=1)
    dots = lax.dot_general(A1, A2, (((1,), (1,)), ((), ())))
    dist = s1[:, None] + s2[None, :] - 2.0 * dots
    iota = lax.broadcasted_iota(jnp.int32, (G, S2), 1)
    INF = jnp.float32(3e38)
    cur = dist
    mvals, idxs = [], []
    for _ in range(3):
        mv = jnp.min(cur, axis=1)
        ij = jnp.min(jnp.where(cur == mv[:, None], iota, S2), axis=1)
        mvals.append(mv)
        idxs.append(ij)
        cur = jnp.where(iota == ij[:, None], INF, cur)
    idx_ref[0] = jnp.stack(idxs, axis=1)
    d_ref[0] = jnp.stack(mvals, axis=1)


def _fp_select(xyz1, xyz2, G):
    B, S1, _ = xyz1.shape
    S2 = xyz2.shape[1]
    return pl.pallas_call(
        functools.partial(_fp_select_body, S2),
        grid=(B, S1 // G),
        in_specs=[
            pl.BlockSpec((1, G, 3), lambda b, s: (b, s, 0)),
            pl.BlockSpec((1, S2, 3), lambda b, s: (b, 0, 0)),
        ],
        out_specs=[
            pl.BlockSpec((1, G, 3), lambda b, s: (b, s, 0)),
            pl.BlockSpec((1, G, 3), lambda b, s: (b, s, 0)),
        ],
        out_shape=[
            jax.ShapeDtypeStruct((B, S1, 3), jnp.int32),
            jax.ShapeDtypeStruct((B, S1, 3), jnp.float32),
        ],
    )(xyz1, xyz2)


def _fp_mlp_body(nl, relus, S2, idx_ref, w_ref, f1_ref, f2_ref, *args):
    w = [args[3 * i] for i in range(nl)]
    a = [args[3 * i + 1] for i in range(nl)]
    d = [args[3 * i + 2] for i in range(nl)]
    out_ref = args[3 * nl]
    idxs = idx_ref[0]
    ws = w_ref[0]
    G = idxs.shape[0]
    iota = lax.broadcasted_iota(jnp.int32, (G, S2), 1)
    f2 = f2_ref[0]
    P = []
    for j in range(3):
        oh = jnp.where(iota == idxs[:, j][:, None], 1.0, 0.0)
        P.append(jnp.dot(oh, f2, preferred_element_type=jnp.float32,
                         precision=lax.Precision.HIGHEST))
    interp = (P[0] * ws[:, 0][:, None] + P[1] * ws[:, 1][:, None]
              + P[2] * ws[:, 2][:, None])
    h = jnp.concatenate([f1_ref[0], interp], axis=1)
    for i in range(nl):
        h = jnp.dot(h, w[i][...], preferred_element_type=jnp.float32)
        h = h * a[i][...] + d[i][...]
        if relus[i]:
            h = jnp.maximum(h, 0.0)
    out_ref[0] = h


def _fp(xyz1, xyz2, feats1, feats2, layers, relus, G):
    B, S1, _ = xyz1.shape
    S2 = xyz2.shape[1]
    C1 = feats1.shape[2]
    C2 = feats2.shape[2]
    nl = len(layers)
    Cout = layers[-1][0].shape[1]
    dists = (jnp.sum(xyz1 ** 2, -1)[:, :, None] + jnp.sum(xyz2 ** 2, -1)[:, None, :]
             - 2.0 * jnp.einsum('bnc,bmc->bnm', xyz1, xyz2))
    idx3 = jnp.argsort(dists, axis=-1)[:, :, :3]
    d3 = jnp.take_along_axis(dists, idx3, axis=-1)
    dist_recip = 1.0 / (d3 + 1e-8)
    norm = jnp.sum(dist_recip, axis=-1, keepdims=True)
    weight = dist_recip / norm
    in_specs = [
        pl.BlockSpec((1, G, 3), lambda b, s: (b, s, 0)),
        pl.BlockSpec((1, G, 3), lambda b, s: (b, s, 0)),
        pl.BlockSpec((1, G, C1), lambda b, s: (b, s, 0)),
        pl.BlockSpec((1, S2, C2), lambda b, s: (b, 0, 0)),
    ]
    ops = []
    for (wt, av, dv) in layers:
        in_specs.append(pl.BlockSpec(wt.shape, lambda b, s: (0, 0)))
        in_specs.append(pl.BlockSpec(av.shape, lambda b, s: (0, 0)))
        in_specs.append(pl.BlockSpec(dv.shape, lambda b, s: (0, 0)))
        ops.extend([wt, av, dv])
    return pl.pallas_call(
        functools.partial(_fp_mlp_body, nl, tuple(relus), S2),
        grid=(B, S1 // G),
        in_specs=in_specs,
        out_specs=pl.BlockSpec((1, G, Cout), lambda b, s: (b, s, 0)),
        out_shape=jax.ShapeDtypeStruct((B, S1, Cout), jnp.float32),
    )(idx3, weight, feats1, feats2, *ops)


def _prep_layers(raw):
    out = []
    for (W, b, gamma, beta, mean, var) in raw:
        a = gamma / jnp.sqrt(var + 1e-5)
        dv = (b - mean) * a + beta
        out.append((W.T, a[None, :], dv[None, :]))
    return out


def _ceil16(x):
    return (x + 15) // 16 * 16


def _sa_stage(xyz, feats, raw_layers, S, radius, K, SB, G):
    B, N, _ = xyz.shape
    C = feats.shape[2]
    fps_idx = _fps(xyz, S)
    new_xyz = jax.vmap(lambda p, i: p[i])(xyz, fps_idx)
    gidx = _ballquery(radius, K, xyz, new_xyz, SB)
    Dp = _ceil16(3 + C)
    tab = jnp.concatenate([xyz, feats], axis=-1)
    tab = jnp.pad(tab, ((0, 0), (0, 0), (0, Dp - (3 + C))))
    tab = tab.reshape(B * N, Dp)
    gathered = _sc_gather(tab, gidx.reshape(-1)).reshape(B * S, K, Dp)
    centers = jnp.pad(new_xyz, ((0, 0), (0, 0), (0, 5))).reshape(B * S, 8)
    layers = _prep_layers(raw_layers)
    w0, a0, d0 = layers[0]
    layers[0] = (jnp.pad(w0, ((0, Dp - w0.shape[0]), (0, 0))), a0, d0)
    new_feats = _sa_mlp(gathered, centers, layers, G).reshape(B, S, -1)
    return new_xyz, new_feats


def kernel(xyz, points, params):
    B, N, _ = xyz.shape
    feats0 = points

    l1_xyz, l1_f = _sa_stage(
        xyz, feats0, params['sa1'], 1024, 0.1, 32, SB=256, G=128)
    l2_xyz, l2_f = _sa_stage(
        l1_xyz, l1_f, params['sa2'], 256, 0.2, 32, SB=256, G=128)
    l3_xyz, l3_f = _sa_stage(
        l2_xyz, l2_f, params['sa3'], 64, 0.4, 32, SB=64, G=64)

    fp3_layers = _prep_layers(params['fp3'])
    l2_f = _fp(l2_xyz, l3_xyz, l2_f, l3_f, fp3_layers,
               [True, True], G=256)
    fp2_layers = _prep_layers(params['fp2'])
    l1_f = _fp(l1_xyz, l2_xyz, l1_f, l2_f, fp2_layers,
               [True, True], G=256)

    h = params['head']
    a1 = h['bn1_gamma'] / jnp.sqrt(h['bn1_var'] + 1e-5)
    d1 = (h['conv1_b'] - h['bn1_mean']) * a1 + h['bn1_beta']
    head_layers = [
        (h['conv1_W'].T, a1[None, :], d1[None, :]),
        (h['conv2_W'].T, jnp.ones((1, h['conv2_W'].shape[0]), jnp.float32),
         h['conv2_b'][None, :]),
    ]
    fp1_layers = _prep_layers(params['fp1']) + head_layers
    out = _fp(xyz, l1_xyz, feats0, l1_f, fp1_layers,
              [True, True, True, True, False], G=512)
    return jnp.transpose(out, (0, 2, 1))


# speedup vs baseline: 4.7465x; 1.1418x over previous
import functools

import jax
import jax.numpy as jnp
from jax import lax
from jax.experimental import pallas as pl
from jax.experimental.pallas import tpu as pltpu
from jax.experimental.pallas import tpu_sc as plsc


def _fps_body(S, x_ref, y_ref, z_ref, idx_ref, cx_ref, cy_ref, cz_ref):
    B, R, L = x_ref.shape
    N = R * L
    lin = (lax.broadcasted_iota(jnp.int32, (B, R, L), 1) * L
           + lax.broadcasted_iota(jnp.int32, (B, R, L), 2))
    X = x_ref[...]
    Y = y_ref[...]
    Z = z_ref[...]

    def red2(v, red):
        return red(red(v, axis=2), axis=1)

    def body(t, carry):
        dist, far = carry
        mask = lin == far[:, None, None]
        cx = red2(jnp.where(mask, X, 0.0), jnp.sum)
        cy = red2(jnp.where(mask, Y, 0.0), jnp.sum)
        cz = red2(jnp.where(mask, Z, 0.0), jnp.sum)
        idx_ref[pl.ds(t, 1), :] = far[None, :]
        cx_ref[pl.ds(t, 1), :] = cx[None, :]
        cy_ref[pl.ds(t, 1), :] = cy[None, :]
        cz_ref[pl.ds(t, 1), :] = cz[None, :]
        d = ((X - cx[:, None, None]) ** 2 + (Y - cy[:, None, None]) ** 2
             + (Z - cz[:, None, None]) ** 2)
        dist = jnp.minimum(dist, d)
        m = red2(dist, jnp.max)
        li = red2(jnp.where(dist == m[:, None, None], lin, N), jnp.min)
        return dist, li

    dist0 = jnp.full((B, R, L), 1e10, dtype=jnp.float32)
    far0 = jnp.zeros((B,), dtype=jnp.int32)
    lax.fori_loop(0, S, body, (dist0, far0))


def _fps(xyz, S):
    B, N, _ = xyz.shape
    R = N // 128
    X = xyz[..., 0].reshape(B, R, 128)
    Y = xyz[..., 1].reshape(B, R, 128)
    Z = xyz[..., 2].reshape(B, R, 128)
    outs = pl.pallas_call(
        functools.partial(_fps_body, S),
        out_shape=[
            jax.ShapeDtypeStruct((S, B), jnp.int32),
            jax.ShapeDtypeStruct((S, B), jnp.float32),
            jax.ShapeDtypeStruct((S, B), jnp.float32),
            jax.ShapeDtypeStruct((S, B), jnp.float32),
        ],
    )(X, Y, Z)
    return outs[0].T


def _ballquery_body(r2, K, N, nx_ref, x_ref, out_ref):
    b = pl.program_id(0)
    A = nx_ref[0]
    X = x_ref[0]
    SB = A.shape[0]
    s2 = jnp.sum(A * A, axis=1)
    n2 = jnp.sum(X * X, axis=1)
    dots = lax.dot_general(A, X, (((1,), (1,)), ((), ())))
    d = s2[:, None] + n2[None, :] - 2.0 * dots
    iota = lax.broadcasted_iota(jnp.int32, (SB, N), 1)
    cand0 = jnp.where(d > r2, N, iota)
    lane = lax.broadcasted_iota(jnp.int32, (SB, K), 1)
    BIG = jnp.int32(1 << 30)

    def body(j, carry):
        cand, cols = carry
        mj = jnp.min(cand, axis=1)
        cols = jnp.where(lane == j, mj[:, None], cols)
        cand = jnp.where(cand == mj[:, None], BIG, cand)
        return cand, cols

    _, cols = lax.fori_loop(0, K, body, (cand0, jnp.zeros((SB, K), jnp.int32)))
    col0 = cols[:, 0:1]
    cols = jnp.where(cols >= N, col0, cols)
    cols = jnp.minimum(cols, N - 1)
    out_ref[0] = cols + b * N


def _ballquery(radius, K, xyz, new_xyz, SB):
    B, N, _ = xyz.shape
    S = new_xyz.shape[1]
    r2 = float(radius) ** 2
    return pl.pallas_call(
        functools.partial(_ballquery_body, r2, K, N),
        grid=(B, S // SB),
        in_specs=[
            pl.BlockSpec((1, SB, 3), lambda b, s: (b, s, 0)),
            pl.BlockSpec((1, N, 3), lambda b, s: (b, 0, 0)),
        ],
        out_specs=pl.BlockSpec((1, SB, K), lambda b, s: (b, s, 0)),
        out_shape=jax.ShapeDtypeStruct((B, S, K), jnp.int32),
    )(new_xyz, xyz)


def _sc_gather(table, idx):
    V, Dp = table.shape
    (Bi,) = idx.shape
    NW = 32
    bpw = Bi // NW
    CH = min(128, bpw)
    nch = bpw // CH
    mesh = plsc.VectorSubcoreMesh(core_axis_name="c", subcore_axis_name="s")

    @functools.partial(
        pl.kernel,
        out_type=jax.ShapeDtypeStruct((Bi, Dp), jnp.float32),
        scratch_types=[
            pltpu.VMEM((CH,), jnp.int32),
            pltpu.VMEM((CH, Dp), jnp.float32),
            pltpu.SemaphoreType.DMA,
        ],
        mesh=mesh,
        compiler_params=pltpu.CompilerParams(use_tc_tiling_on_sc=False),
    )
    def k(table_hbm, idx_hbm, out_hbm, idx_v, rows_v, sem):
        wid = lax.axis_index("s") * 2 + lax.axis_index("c")
        base = wid * bpw

        def body(c, carry):
            off = base + c * CH
            pltpu.sync_copy(idx_hbm.at[pl.ds(off, CH)], idx_v)
            pltpu.async_copy(table_hbm.at[idx_v], rows_v, sem).wait()
            pltpu.sync_copy(rows_v, out_hbm.at[pl.ds(off, CH)])
            return carry

        lax.fori_loop(0, nch, body, 0)

    return k(table, idx)


def _sa_mlp_body(nl, G, K, Dp, gath_ref, cen_ref, *args):
    w = [args[3 * i] for i in range(nl)]
    a = [args[3 * i + 1] for i in range(nl)]
    d = [args[3 * i + 2] for i in range(nl)]
    out_ref = args[3 * nl]
    g = gath_ref[...]
    c = cen_ref[...]
    cpad = jnp.concatenate(
        [c[:, :3].reshape(G, 1, 3), jnp.zeros((G, 1, Dp - 3), jnp.float32)],
        axis=2)
    h = (g - cpad).reshape(G * K, Dp)
    for i in range(nl):
        h = jnp.dot(h, w[i][...], preferred_element_type=jnp.float32)
        h = jnp.maximum(h * a[i][...] + d[i][...], 0.0)
    C = h.shape[-1]
    out_ref[...] = jnp.max(h.reshape(G, K, C), axis=1)


def _sa_mlp(gathered, centers, layers, G):
    BS, K, Dp = gathered.shape
    nl = len(layers)
    Cout = layers[-1][0].shape[1]
    in_specs = [
        pl.BlockSpec((G, K, Dp), lambda i: (i, 0, 0)),
        pl.BlockSpec((G, 8), lambda i: (i, 0)),
    ]
    ops = []
    for (wt, av, dv) in layers:
        in_specs.append(pl.BlockSpec(wt.shape, lambda i: (0, 0)))
        in_specs.append(pl.BlockSpec(av.shape, lambda i: (0, 0)))
        in_specs.append(pl.BlockSpec(dv.shape, lambda i: (0, 0)))
        ops.extend([wt, av, dv])
    return pl.pallas_call(
        functools.partial(_sa_mlp_body, nl, G, K, Dp),
        grid=(BS // G,),
        in_specs=in_specs,
        out_specs=pl.BlockSpec((G, Cout), lambda i: (i, 0)),
        out_shape=jax.ShapeDtypeStruct((BS, Cout), jnp.float32),
    )(gathered, centers, *ops)


def _fp_select_body(S2, x1_ref, x2_ref, idx_ref, d_ref):
    A1 = x1_ref[0]
    A2 = x2_ref[0]
    G = A1.shape[0]
    s1 = jnp.sum(A1 * A1, axis=1)
    s2 = jnp.sum(A2 * A2, axis=1)
    dots = lax.dot_general(A1, A2, (((1,), (1,)), ((), ())))
    dist = s1[:, None] + s2[None, :] - 2.0 * dots
    iota = lax.broadcasted_iota(jnp.int32, (G, S2), 1)
    INF = jnp.float32(3e38)
    cur = dist
    mvals, idxs = [], []
    for _ in range(3):
        mv = jnp.min(cur, axis=1)
        ij = jnp.min(jnp.where(cur == mv[:, None], iota, S2), axis=1)
        mvals.append(mv)
        idxs.append(ij)
        cur = jnp.where(iota == ij[:, None], INF, cur)
    idx_ref[0] = jnp.stack(idxs, axis=1)
    d_ref[0] = jnp.stack(mvals, axis=1)


def _fp_select(xyz1, xyz2, G):
    B, S1, _ = xyz1.shape
    S2 = xyz2.shape[1]
    return pl.pallas_call(
        functools.partial(_fp_select_body, S2),
        grid=(B, S1 // G),
        in_specs=[
            pl.BlockSpec((1, G, 3), lambda b, s: (b, s, 0)),
            pl.BlockSpec((1, S2, 3), lambda b, s: (b, 0, 0)),
        ],
        out_specs=[
            pl.BlockSpec((1, G, 3), lambda b, s: (b, s, 0)),
            pl.BlockSpec((1, G, 3), lambda b, s: (b, s, 0)),
        ],
        out_shape=[
            jax.ShapeDtypeStruct((B, S1, 3), jnp.int32),
            jax.ShapeDtypeStruct((B, S1, 3), jnp.float32),
        ],
    )(xyz1, xyz2)


def _fp_mlp_body(nl, relus, S2, idx_ref, w_ref, f1_ref, f2_ref, *args):
    w = [args[3 * i] for i in range(nl)]
    a = [args[3 * i + 1] for i in range(nl)]
    d = [args[3 * i + 2] for i in range(nl)]
    out_ref = args[3 * nl]
    idxs = idx_ref[0]
    ws = w_ref[0]
    G = idxs.shape[0]
    iota = lax.broadcasted_iota(jnp.int32, (G, S2), 1)
    f2 = f2_ref[0]
    P = []
    for j in range(3):
        oh = jnp.where(iota == idxs[:, j][:, None], 1.0, 0.0)
        P.append(jnp.dot(oh, f2, preferred_element_type=jnp.float32,
                         precision=lax.Precision.HIGHEST))
    interp = (P[0] * ws[:, 0][:, None] + P[1] * ws[:, 1][:, None]
              + P[2] * ws[:, 2][:, None])
    h = jnp.concatenate([f1_ref[0], interp], axis=1)
    for i in range(nl):
        h = jnp.dot(h, w[i][...], preferred_element_type=jnp.float32)
        h = h * a[i][...] + d[i][...]
        if relus[i]:
            h = jnp.maximum(h, 0.0)
    out_ref[0] = h


def _fp(xyz1, xyz2, feats1, feats2, layers, relus, G):
    B, S1, _ = xyz1.shape
    S2 = xyz2.shape[1]
    C1 = feats1.shape[2]
    C2 = feats2.shape[2]
    nl = len(layers)
    Cout = layers[-1][0].shape[1]
    dists = (jnp.sum(xyz1 ** 2, -1)[:, :, None] + jnp.sum(xyz2 ** 2, -1)[:, None, :]
             - 2.0 * jnp.einsum('bnc,bmc->bnm', xyz1, xyz2))
    _, idx3 = lax.top_k(-dists, 3)
    d3 = jnp.take_along_axis(dists, idx3, axis=-1)
    dist_recip = 1.0 / (d3 + 1e-8)
    norm = jnp.sum(dist_recip, axis=-1, keepdims=True)
    weight = dist_recip / norm
    in_specs = [
        pl.BlockSpec((1, G, 3), lambda b, s: (b, s, 0)),
        pl.BlockSpec((1, G, 3), lambda b, s: (b, s, 0)),
        pl.BlockSpec((1, G, C1), lambda b, s: (b, s, 0)),
        pl.BlockSpec((1, S2, C2), lambda b, s: (b, 0, 0)),
    ]
    ops = []
    for (wt, av, dv) in layers:
        in_specs.append(pl.BlockSpec(wt.shape, lambda b, s: (0, 0)))
        in_specs.append(pl.BlockSpec(av.shape, lambda b, s: (0, 0)))
        in_specs.append(pl.BlockSpec(dv.shape, lambda b, s: (0, 0)))
        ops.extend([wt, av, dv])
    return pl.pallas_call(
        functools.partial(_fp_mlp_body, nl, tuple(relus), S2),
        grid=(B, S1 // G),
        in_specs=in_specs,
        out_specs=pl.BlockSpec((1, G, Cout), lambda b, s: (b, s, 0)),
        out_shape=jax.ShapeDtypeStruct((B, S1, Cout), jnp.float32),
    )(idx3, weight, feats1, feats2, *ops)


def _prep_layers(raw):
    out = []
    for (W, b, gamma, beta, mean, var) in raw:
        a = gamma / jnp.sqrt(var + 1e-5)
        dv = (b - mean) * a + beta
        out.append((W.T, a[None, :], dv[None, :]))
    return out


def _ceil16(x):
    return (x + 15) // 16 * 16


def _sa_stage(xyz, feats, raw_layers, S, radius, K, SB, G):
    B, N, _ = xyz.shape
    C = feats.shape[2]
    fps_idx = _fps(xyz, S)
    new_xyz = jax.vmap(lambda p, i: p[i])(xyz, fps_idx)
    gidx = _ballquery(radius, K, xyz, new_xyz, SB)
    Dp = _ceil16(3 + C)
    tab = jnp.concatenate([xyz, feats], axis=-1)
    tab = jnp.pad(tab, ((0, 0), (0, 0), (0, Dp - (3 + C))))
    tab = tab.reshape(B * N, Dp)
    gathered = _sc_gather(tab, gidx.reshape(-1)).reshape(B * S, K, Dp)
    centers = jnp.pad(new_xyz, ((0, 0), (0, 0), (0, 5))).reshape(B * S, 8)
    layers = _prep_layers(raw_layers)
    w0, a0, d0 = layers[0]
    layers[0] = (jnp.pad(w0, ((0, Dp - w0.shape[0]), (0, 0))), a0, d0)
    new_feats = _sa_mlp(gathered, centers, layers, G).reshape(B, S, -1)
    return new_xyz, new_feats


def kernel(xyz, points, params):
    B, N, _ = xyz.shape
    feats0 = points

    l1_xyz, l1_f = _sa_stage(
        xyz, feats0, params['sa1'], 1024, 0.1, 32, SB=256, G=128)
    l2_xyz, l2_f = _sa_stage(
        l1_xyz, l1_f, params['sa2'], 256, 0.2, 32, SB=256, G=128)
    l3_xyz, l3_f = _sa_stage(
        l2_xyz, l2_f, params['sa3'], 64, 0.4, 32, SB=64, G=64)

    fp3_layers = _prep_layers(params['fp3'])
    l2_f = _fp(l2_xyz, l3_xyz, l2_f, l3_f, fp3_layers,
               [True, True], G=256)
    fp2_layers = _prep_layers(params['fp2'])
    l1_f = _fp(l1_xyz, l2_xyz, l1_f, l2_f, fp2_layers,
               [True, True], G=256)

    h = params['head']
    a1 = h['bn1_gamma'] / jnp.sqrt(h['bn1_var'] + 1e-5)
    d1 = (h['conv1_b'] - h['bn1_mean']) * a1 + h['bn1_beta']
    head_layers = [
        (h['conv1_W'].T, a1[None, :], d1[None, :]),
        (h['conv2_W'].T, jnp.ones((1, h['conv2_W'].shape[0]), jnp.float32),
         h['conv2_b'][None, :]),
    ]
    fp1_layers = _prep_layers(params['fp1']) + head_layers
    out = _fp(xyz, l1_xyz, feats0, l1_f, fp1_layers,
              [True, True, True, True, False], G=512)
    return jnp.transpose(out, (0, 2, 1))
